# Initial kernel scaffold; baseline (speedup 1.0000x reference)
import jax
import jax.numpy as jnp
from jax.experimental import pallas as pl


def kernel(x, edge_index, W1, att_src1, att_dst1, bias1, W2, att_src2, att_dst2, bias2):
    raise NotImplementedError("write your pallas kernel here")


# speedup vs baseline: 28.9400x; 28.9400x over previous
import functools

import jax
import jax.numpy as jnp
from jax import lax
from jax.experimental import pallas as pl
from jax.experimental.pallas import tpu as pltpu
from jax.experimental.pallas import tpu_sc as plsc

N_NODES = 10000
LATENT = 128
N_EDGES = 320000
NC = 2
NS = 16
NW = NC * NS
EPT = N_EDGES // NW
CHUNK = 80
NCHUNKS = EPT // CHUNK
NPAD = 10240
ROWS_PT = NPAD // NS
BR = 2000
GRID_R = N_NODES // BR

_mesh = plsc.VectorSubcoreMesh(core_axis_name="c", subcore_axis_name="s")


_BCAST_DNUMS = lax.GatherDimensionNumbers(
    offset_dims=(), collapsed_slice_dims=(0,), start_index_map=(0,))


def _lane_bcast(vec16, lane):
    idx = jnp.full((16, 1), lane, dtype=jnp.int32)
    return lax.gather(vec16, idx, _BCAST_DNUMS, (1,),
                      mode=lax.GatherScatterMode.PROMISE_IN_BOUNDS)


@functools.partial(
    pl.kernel,
    mesh=_mesh,
    compiler_params=pltpu.CompilerParams(needs_layout_passes=False),
    out_type=[
        jax.ShapeDtypeStruct((NC, NPAD, LATENT), jnp.float32),
        jax.ShapeDtypeStruct((NC, NPAD), jnp.float32),
    ],
    scratch_types=[
        pltpu.VMEM((2, CHUNK), jnp.int32),
        pltpu.VMEM((CHUNK,), jnp.float32),
        pltpu.VMEM((CHUNK,), jnp.float32),
        pltpu.VMEM((CHUNK,), jnp.float32),
        pltpu.VMEM((CHUNK, LATENT), jnp.float32),
        pltpu.VMEM_SHARED((NPAD, LATENT), jnp.float32),
        pltpu.VMEM_SHARED((NPAD,), jnp.float32),
        pltpu.SemaphoreType.DMA,
        pltpu.SemaphoreType.DMA,
        pltpu.SemaphoreType.DMA,
        pltpu.SemaphoreType.DMA,
        pltpu.SemaphoreType.DMA,
    ],
)
def _sc_edge_aggregate(h_hbm, asrc_hbm, adst_hbm, sd_hbm, zs_hbm, zd_hbm,
                       s_out, d_out,
                       sd_v, ae_v, be_v, wv_v, rows_v, s_sh, d_sh,
                       sem_a, sem_b, sem_r, sem_s, sem_w):
    c = lax.axis_index("c")
    s = lax.axis_index("s")
    wid = c * NS + s
    row0 = s * ROWS_PT

    pltpu.sync_copy(zs_hbm.at[pl.ds(row0, ROWS_PT)],
                    s_sh.at[pl.ds(row0, ROWS_PT)])

    @pl.when(s == 0)
    def _():
        pltpu.sync_copy(zd_hbm, d_sh)

    plsc.subcore_barrier()

    def chunk_body(j, carry):
        pltpu.sync_copy(sd_hbm.at[wid, j], sd_v)
        ca = pltpu.async_copy(asrc_hbm.at[sd_v.at[0]], ae_v, sem_a)
        cb = pltpu.async_copy(adst_hbm.at[sd_v.at[1]], be_v, sem_b)
        cr = pltpu.async_copy(h_hbm.at[sd_v.at[0]], rows_v, sem_r)
        ca.wait()
        cb.wait()
        for g in range(CHUNK // 16):
            sl = pl.ds(g * 16, 16)
            a = ae_v[sl] + be_v[sl]
            a = jnp.where(a >= 0.0, a, 0.2 * a)
            wv_v[sl] = jnp.exp(a)
        cr.wait()

        def scale16(q, carry2):
            wgrp = wv_v[pl.ds(q * 16, 16)]
            r0 = q * 16
            for rr in range(16):
                wb = _lane_bcast(wgrp, rr)
                for g2 in range(LATENT // 16):
                    sl2 = pl.ds(g2 * 16, 16)
                    rows_v[r0 + rr, sl2] = rows_v[r0 + rr, sl2] * wb
            return carry2

        lax.fori_loop(0, CHUNK // 16, scale16, 0)

        cs = pltpu.async_copy(rows_v, s_sh.at[sd_v.at[1]], sem_s, add=True)
        cw = pltpu.async_copy(wv_v, d_sh.at[sd_v.at[1]], sem_w, add=True)
        cs.wait()
        cw.wait()
        return carry

    lax.fori_loop(0, NCHUNKS, chunk_body, 0)

    plsc.subcore_barrier()

    pltpu.sync_copy(s_sh.at[pl.ds(row0, ROWS_PT)],
                    s_out.at[c, pl.ds(row0, ROWS_PT)])

    @pl.when(s == 0)
    def _():
        pltpu.sync_copy(d_sh, d_out.at[c])


def _lin_body(x_ref, w_ref, att_ref, h_ref, ab_ref):
    h = jnp.dot(x_ref[...], w_ref[...], preferred_element_type=jnp.float32)
    h_ref[...] = h
    ab_ref[...] = jnp.dot(h, att_ref[...], preferred_element_type=jnp.float32)


def _lin_call(x, W, att):
    return pl.pallas_call(
        _lin_body,
        grid=(GRID_R,),
        in_specs=[
            pl.BlockSpec((BR, LATENT), lambda i: (i, 0)),
            pl.BlockSpec((LATENT, LATENT), lambda i: (0, 0)),
            pl.BlockSpec((LATENT, 2), lambda i: (0, 0)),
        ],
        out_specs=[
            pl.BlockSpec((BR, LATENT), lambda i: (i, 0)),
            pl.BlockSpec((BR, 2), lambda i: (i, 0)),
        ],
        out_shape=[
            jax.ShapeDtypeStruct((N_NODES, LATENT), jnp.float32),
            jax.ShapeDtypeStruct((N_NODES, 2), jnp.float32),
        ],
    )(x, W, att)


def _combine(s_ref, dt_ref, h_ref, ab_ref, bias_ref):
    ab = ab_ref[...]
    e = ab[:, 0:1] + ab[:, 1:2]
    e = jnp.where(e >= 0.0, e, 0.2 * e)
    wself = jnp.exp(e)
    h = h_ref[...]
    ssum = s_ref[0] + s_ref[1] + wself * h
    dt = dt_ref[...]
    den = dt[:, 0:1] + dt[:, 1:2] + wself
    out = ssum / den + bias_ref[...]
    return jnp.where(out > 0.0, out, jnp.exp(out) - 1.0)


def _mid_body(s_ref, dt_ref, h_ref, ab_ref, bias_ref, w2_ref, att2_ref,
              z_ref, h2_ref, ab2_ref):
    z = _combine(s_ref, dt_ref, h_ref, ab_ref, bias_ref)
    z_ref[...] = z
    h2 = jnp.dot(z, w2_ref[...], preferred_element_type=jnp.float32)
    h2_ref[...] = h2
    ab2_ref[...] = jnp.dot(h2, att2_ref[...], preferred_element_type=jnp.float32)


def _mid_call(S, Dt, h, ab, bias, W2, att2):
    return pl.pallas_call(
        _mid_body,
        grid=(GRID_R,),
        in_specs=[
            pl.BlockSpec((NC, BR, LATENT), lambda i: (0, i, 0)),
            pl.BlockSpec((BR, NC), lambda i: (i, 0)),
            pl.BlockSpec((BR, LATENT), lambda i: (i, 0)),
            pl.BlockSpec((BR, 2), lambda i: (i, 0)),
            pl.BlockSpec((1, LATENT), lambda i: (0, 0)),
            pl.BlockSpec((LATENT, LATENT), lambda i: (0, 0)),
            pl.BlockSpec((LATENT, 2), lambda i: (0, 0)),
        ],
        out_specs=[
            pl.BlockSpec((BR, LATENT), lambda i: (i, 0)),
            pl.BlockSpec((BR, LATENT), lambda i: (i, 0)),
            pl.BlockSpec((BR, 2), lambda i: (i, 0)),
        ],
        out_shape=[
            jax.ShapeDtypeStruct((N_NODES, LATENT), jnp.float32),
            jax.ShapeDtypeStruct((N_NODES, LATENT), jnp.float32),
            jax.ShapeDtypeStruct((N_NODES, 2), jnp.float32),
        ],
    )(S, Dt, h, ab, bias, W2, att2)


def _fin_body(s_ref, dt_ref, h_ref, ab_ref, bias_ref, out_ref):
    out_ref[...] = _combine(s_ref, dt_ref, h_ref, ab_ref, bias_ref)


def _fin_call(S, Dt, h, ab, bias):
    return pl.pallas_call(
        _fin_body,
        grid=(GRID_R,),
        in_specs=[
            pl.BlockSpec((NC, BR, LATENT), lambda i: (0, i, 0)),
            pl.BlockSpec((BR, NC), lambda i: (i, 0)),
            pl.BlockSpec((BR, LATENT), lambda i: (i, 0)),
            pl.BlockSpec((BR, 2), lambda i: (i, 0)),
            pl.BlockSpec((1, LATENT), lambda i: (0, 0)),
        ],
        out_specs=pl.BlockSpec((BR, LATENT), lambda i: (i, 0)),
        out_shape=jax.ShapeDtypeStruct((N_NODES, LATENT), jnp.float32),
    )(S, Dt, h, ab, bias)


def kernel(x, edge_index, W1, att_src1, att_dst1, bias1,
           W2, att_src2, att_dst2, bias2):
    ei = edge_index.astype(jnp.int32)
    sd_rs = jnp.stack([ei[0].reshape(NW, NCHUNKS, CHUNK),
                       ei[1].reshape(NW, NCHUNKS, CHUNK)], axis=2)
    att1 = jnp.stack([att_src1, att_dst1], axis=1)
    att2 = jnp.stack([att_src2, att_dst2], axis=1)
    b1 = bias1.reshape(1, LATENT)
    b2 = bias2.reshape(1, LATENT)
    zs = jnp.zeros((NPAD, LATENT), jnp.float32)
    zd = jnp.zeros((NPAD,), jnp.float32)

    h1, ab1 = _lin_call(x, W1, att1)
    asrc1 = ab1[:, 0].ravel()
    adst1 = ab1[:, 1].ravel()
    S1, Dn1 = _sc_edge_aggregate(h1, asrc1, adst1, sd_rs, zs, zd)
    z, h2, ab2 = _mid_call(S1, Dn1.T, h1, ab1, b1, W2, att2)
    asrc2 = ab2[:, 0].ravel()
    adst2 = ab2[:, 1].ravel()
    S2, Dn2 = _sc_edge_aggregate(h2, asrc2, adst2, sd_rs, zs, zd)
    xbar = _fin_call(S2, Dn2.T, h2, ab2, b2)
    return (xbar, z)


# speedup vs baseline: 42.7627x; 1.4776x over previous
import functools

import jax
import jax.numpy as jnp
from jax import lax
from jax.experimental import pallas as pl
from jax.experimental.pallas import tpu as pltpu
from jax.experimental.pallas import tpu_sc as plsc

N_NODES = 10000
LATENT = 128
N_EDGES = 320000
NC = 2
NS = 16
NW = NC * NS
EPT = N_EDGES // NW
CHUNK = 80
NCHUNKS = EPT // CHUNK
NPAD = 10240
ROWS_PT = NPAD // NS
BR = 2000
GRID_R = N_NODES // BR

_mesh = plsc.VectorSubcoreMesh(core_axis_name="c", subcore_axis_name="s")


_BCAST_DNUMS = lax.GatherDimensionNumbers(
    offset_dims=(), collapsed_slice_dims=(0,), start_index_map=(0,))


def _lane_bcast(vec16, lane):
    idx = jnp.full((16, 1), lane, dtype=jnp.int32)
    return lax.gather(vec16, idx, _BCAST_DNUMS, (1,),
                      mode=lax.GatherScatterMode.PROMISE_IN_BOUNDS)


GRP = 4
NGRP = NCHUNKS // GRP


@functools.partial(
    pl.kernel,
    mesh=_mesh,
    compiler_params=pltpu.CompilerParams(needs_layout_passes=False),
    out_type=[
        jax.ShapeDtypeStruct((NC, NPAD, LATENT), jnp.float32),
        jax.ShapeDtypeStruct((NC, NPAD), jnp.float32),
    ],
    scratch_types=[
        pltpu.VMEM((GRP, 2, CHUNK), jnp.int32),
        pltpu.VMEM((CHUNK,), jnp.float32),
        pltpu.VMEM((CHUNK,), jnp.float32),
        pltpu.VMEM((CHUNK,), jnp.float32),
        pltpu.VMEM((CHUNK,), jnp.float32),
        pltpu.VMEM((CHUNK,), jnp.float32),
        pltpu.VMEM((CHUNK,), jnp.float32),
        pltpu.VMEM((CHUNK, LATENT), jnp.float32),
        pltpu.VMEM((CHUNK, LATENT), jnp.float32),
        pltpu.VMEM_SHARED((NPAD, LATENT), jnp.float32),
        pltpu.VMEM_SHARED((NPAD,), jnp.float32),
    ] + [pltpu.SemaphoreType.DMA] * 10,
)
def _sc_edge_aggregate(h_hbm, asrc_hbm, adst_hbm, sd_hbm, zs_hbm, zd_hbm,
                       s_out, d_out,
                       sd_blk, ae_a, ae_b, be_a, be_b, wv_a, wv_b,
                       rows_a, rows_b, s_sh, d_sh,
                       sem_aa, sem_ab, sem_ba, sem_bb, sem_ra, sem_rb,
                       sem_sa, sem_sb, sem_wa, sem_wb):
    c = lax.axis_index("c")
    s = lax.axis_index("s")
    wid = c * NS + s
    row0 = s * ROWS_PT

    pltpu.sync_copy(zs_hbm.at[pl.ds(row0, ROWS_PT)],
                    s_sh.at[pl.ds(row0, ROWS_PT)])

    @pl.when(s == 0)
    def _():
        pltpu.sync_copy(zd_hbm, d_sh)

    plsc.subcore_barrier()

    def _issue(i, ae, be, rows, sa, sb, sr):
        ca = pltpu.async_copy(asrc_hbm.at[sd_blk.at[i, 0]], ae, sa)
        cb = pltpu.async_copy(adst_hbm.at[sd_blk.at[i, 1]], be, sb)
        cr = pltpu.async_copy(h_hbm.at[sd_blk.at[i, 0]], rows, sr)
        return ca, cb, cr

    def _compute(g3, ae, be, wv, rows):
        ca, cb, cr = g3
        ca.wait()
        cb.wait()
        for g in range(CHUNK // 16):
            sl = pl.ds(g * 16, 16)
            a = ae[sl] + be[sl]
            a = jnp.where(a >= 0.0, a, 0.2 * a)
            wv[sl] = jnp.exp(a)
        cr.wait()

        def scale16(q, carry2):
            wgrp = wv[pl.ds(q * 16, 16)]
            r0 = q * 16
            for rr in range(16):
                wb = _lane_bcast(wgrp, rr)
                for g2 in range(LATENT // 16):
                    sl2 = pl.ds(g2 * 16, 16)
                    rows[r0 + rr, sl2] = rows[r0 + rr, sl2] * wb
            return carry2

        lax.fori_loop(0, CHUNK // 16, scale16, 0)

    def _scatter(i, rows, wv, ss, sw):
        cs = pltpu.async_copy(rows, s_sh.at[sd_blk.at[i, 1]], ss, add=True)
        cw = pltpu.async_copy(wv, d_sh.at[sd_blk.at[i, 1]], sw, add=True)
        return cs, cw

    def _drain_tail_scatters():
        pltpu.make_async_copy(rows_a, s_sh.at[sd_blk.at[2, 1]], sem_sa).wait()
        pltpu.make_async_copy(wv_a, d_sh.at[sd_blk.at[2, 1]], sem_wa).wait()
        pltpu.make_async_copy(rows_b, s_sh.at[sd_blk.at[3, 1]], sem_sb).wait()
        pltpu.make_async_copy(wv_b, d_sh.at[sd_blk.at[3, 1]], sem_wb).wait()

    def group_body(k, carry):
        @pl.when(k > 0)
        def _():
            _drain_tail_scatters()

        pltpu.sync_copy(sd_hbm.at[wid, pl.ds(GRP * k, GRP)], sd_blk)
        ga = _issue(0, ae_a, be_a, rows_a, sem_aa, sem_ba, sem_ra)
        gb = _issue(1, ae_b, be_b, rows_b, sem_ab, sem_bb, sem_rb)

        _compute(ga, ae_a, be_a, wv_a, rows_a)
        csa, cwa = _scatter(0, rows_a, wv_a, sem_sa, sem_wa)
        csa.wait()
        cwa.wait()
        ga2 = _issue(2, ae_a, be_a, rows_a, sem_aa, sem_ba, sem_ra)

        _compute(gb, ae_b, be_b, wv_b, rows_b)
        csb, cwb = _scatter(1, rows_b, wv_b, sem_sb, sem_wb)
        csb.wait()
        cwb.wait()
        gb2 = _issue(3, ae_b, be_b, rows_b, sem_ab, sem_bb, sem_rb)

        _compute(ga2, ae_a, be_a, wv_a, rows_a)
        _scatter(2, rows_a, wv_a, sem_sa, sem_wa)
        _compute(gb2, ae_b, be_b, wv_b, rows_b)
        _scatter(3, rows_b, wv_b, sem_sb, sem_wb)
        return carry

    lax.fori_loop(0, NGRP, group_body, 0)

    _drain_tail_scatters()
    pltpu.sync_copy(sd_hbm.at[wid, NCHUNKS - 1], sd_blk.at[0])
    ga = _issue(0, ae_a, be_a, rows_a, sem_aa, sem_ba, sem_ra)
    _compute(ga, ae_a, be_a, wv_a, rows_a)
    csa, cwa = _scatter(0, rows_a, wv_a, sem_sa, sem_wa)
    csa.wait()
    cwa.wait()

    plsc.subcore_barrier()

    pltpu.sync_copy(s_sh.at[pl.ds(row0, ROWS_PT)],
                    s_out.at[c, pl.ds(row0, ROWS_PT)])

    @pl.when(s == 0)
    def _():
        pltpu.sync_copy(d_sh, d_out.at[c])


def _lin_body(x_ref, w_ref, att_ref, h_ref, ab_ref):
    h = jnp.dot(x_ref[...], w_ref[...], preferred_element_type=jnp.float32)
    h_ref[...] = h
    ab_ref[...] = jnp.dot(h, att_ref[...], preferred_element_type=jnp.float32)


def _lin_call(x, W, att):
    return pl.pallas_call(
        _lin_body,
        grid=(GRID_R,),
        in_specs=[
            pl.BlockSpec((BR, LATENT), lambda i: (i, 0)),
            pl.BlockSpec((LATENT, LATENT), lambda i: (0, 0)),
            pl.BlockSpec((LATENT, 2), lambda i: (0, 0)),
        ],
        out_specs=[
            pl.BlockSpec((BR, LATENT), lambda i: (i, 0)),
            pl.BlockSpec((BR, 2), lambda i: (i, 0)),
        ],
        out_shape=[
            jax.ShapeDtypeStruct((N_NODES, LATENT), jnp.float32),
            jax.ShapeDtypeStruct((N_NODES, 2), jnp.float32),
        ],
    )(x, W, att)


def _combine(s_ref, dt_ref, h_ref, ab_ref, bias_ref):
    ab = ab_ref[...]
    e = ab[:, 0:1] + ab[:, 1:2]
    e = jnp.where(e >= 0.0, e, 0.2 * e)
    wself = jnp.exp(e)
    h = h_ref[...]
    ssum = s_ref[0] + s_ref[1] + wself * h
    dt = dt_ref[...]
    den = dt[:, 0:1] + dt[:, 1:2] + wself
    out = ssum / den + bias_ref[...]
    return jnp.where(out > 0.0, out, jnp.exp(out) - 1.0)


def _mid_body(s_ref, dt_ref, h_ref, ab_ref, bias_ref, w2_ref, att2_ref,
              z_ref, h2_ref, ab2_ref):
    z = _combine(s_ref, dt_ref, h_ref, ab_ref, bias_ref)
    z_ref[...] = z
    h2 = jnp.dot(z, w2_ref[...], preferred_element_type=jnp.float32)
    h2_ref[...] = h2
    ab2_ref[...] = jnp.dot(h2, att2_ref[...], preferred_element_type=jnp.float32)


def _mid_call(S, Dt, h, ab, bias, W2, att2):
    return pl.pallas_call(
        _mid_body,
        grid=(GRID_R,),
        in_specs=[
            pl.BlockSpec((NC, BR, LATENT), lambda i: (0, i, 0)),
            pl.BlockSpec((BR, NC), lambda i: (i, 0)),
            pl.BlockSpec((BR, LATENT), lambda i: (i, 0)),
            pl.BlockSpec((BR, 2), lambda i: (i, 0)),
            pl.BlockSpec((1, LATENT), lambda i: (0, 0)),
            pl.BlockSpec((LATENT, LATENT), lambda i: (0, 0)),
            pl.BlockSpec((LATENT, 2), lambda i: (0, 0)),
        ],
        out_specs=[
            pl.BlockSpec((BR, LATENT), lambda i: (i, 0)),
            pl.BlockSpec((BR, LATENT), lambda i: (i, 0)),
            pl.BlockSpec((BR, 2), lambda i: (i, 0)),
        ],
        out_shape=[
            jax.ShapeDtypeStruct((N_NODES, LATENT), jnp.float32),
            jax.ShapeDtypeStruct((N_NODES, LATENT), jnp.float32),
            jax.ShapeDtypeStruct((N_NODES, 2), jnp.float32),
        ],
    )(S, Dt, h, ab, bias, W2, att2)


def _fin_body(s_ref, dt_ref, h_ref, ab_ref, bias_ref, out_ref):
    out_ref[...] = _combine(s_ref, dt_ref, h_ref, ab_ref, bias_ref)


def _fin_call(S, Dt, h, ab, bias):
    return pl.pallas_call(
        _fin_body,
        grid=(GRID_R,),
        in_specs=[
            pl.BlockSpec((NC, BR, LATENT), lambda i: (0, i, 0)),
            pl.BlockSpec((BR, NC), lambda i: (i, 0)),
            pl.BlockSpec((BR, LATENT), lambda i: (i, 0)),
            pl.BlockSpec((BR, 2), lambda i: (i, 0)),
            pl.BlockSpec((1, LATENT), lambda i: (0, 0)),
        ],
        out_specs=pl.BlockSpec((BR, LATENT), lambda i: (i, 0)),
        out_shape=jax.ShapeDtypeStruct((N_NODES, LATENT), jnp.float32),
    )(S, Dt, h, ab, bias)


def kernel(x, edge_index, W1, att_src1, att_dst1, bias1,
           W2, att_src2, att_dst2, bias2):
    ei = edge_index.astype(jnp.int32)
    sd_rs = jnp.stack([ei[0].reshape(NW, NCHUNKS, CHUNK),
                       ei[1].reshape(NW, NCHUNKS, CHUNK)], axis=2)
    att1 = jnp.stack([att_src1, att_dst1], axis=1)
    att2 = jnp.stack([att_src2, att_dst2], axis=1)
    b1 = bias1.reshape(1, LATENT)
    b2 = bias2.reshape(1, LATENT)
    zs = jnp.zeros((NPAD, LATENT), jnp.float32)
    zd = jnp.zeros((NPAD,), jnp.float32)

    h1, ab1 = _lin_call(x, W1, att1)
    asrc1 = ab1[:, 0].ravel()
    adst1 = ab1[:, 1].ravel()
    S1, Dn1 = _sc_edge_aggregate(h1, asrc1, adst1, sd_rs, zs, zd)
    z, h2, ab2 = _mid_call(S1, Dn1.T, h1, ab1, b1, W2, att2)
    asrc2 = ab2[:, 0].ravel()
    adst2 = ab2[:, 1].ravel()
    S2, Dn2 = _sc_edge_aggregate(h2, asrc2, adst2, sd_rs, zs, zd)
    xbar = _fin_call(S2, Dn2.T, h2, ab2, b2)
    return (xbar, z)


# speedup vs baseline: 57.6805x; 1.3489x over previous
import functools

import jax
import jax.numpy as jnp
from jax import lax
from jax.experimental import pallas as pl
from jax.experimental.pallas import tpu as pltpu
from jax.experimental.pallas import tpu_sc as plsc

N_NODES = 10000
LATENT = 128
N_EDGES = 320000
NC = 2
NS = 16
NW = NC * NS
EPT = N_EDGES // NW
CHUNK = 80
NCHUNKS = EPT // CHUNK
NPAD = 10240
ROWS_PT = NPAD // NS
BR = 2000
GRID_R = N_NODES // BR

_mesh = plsc.VectorSubcoreMesh(core_axis_name="c", subcore_axis_name="s")


_BCAST_DNUMS = lax.GatherDimensionNumbers(
    offset_dims=(), collapsed_slice_dims=(0,), start_index_map=(0,))


def _lane_bcast(vec16, lane):
    idx = jnp.full((16, 1), lane, dtype=jnp.int32)
    return lax.gather(vec16, idx, _BCAST_DNUMS, (1,),
                      mode=lax.GatherScatterMode.PROMISE_IN_BOUNDS)


NBODY = 20


@functools.partial(
    pl.kernel,
    mesh=_mesh,
    compiler_params=pltpu.CompilerParams(needs_layout_passes=False),
    out_type=[
        jax.ShapeDtypeStruct((NC, NPAD, LATENT), jnp.float32),
        jax.ShapeDtypeStruct((NC, NPAD), jnp.float32),
    ],
    scratch_types=[
        pltpu.VMEM((3, 2, CHUNK), jnp.int32),
        pltpu.VMEM((3, 2, CHUNK), jnp.int32),
        pltpu.VMEM((CHUNK,), jnp.float32),
        pltpu.VMEM((CHUNK,), jnp.float32),
        pltpu.VMEM((CHUNK,), jnp.float32),
        pltpu.VMEM((CHUNK,), jnp.float32),
        pltpu.VMEM((CHUNK,), jnp.float32),
        pltpu.VMEM((CHUNK,), jnp.float32),
        pltpu.VMEM((CHUNK,), jnp.float32),
        pltpu.VMEM((CHUNK,), jnp.float32),
        pltpu.VMEM((CHUNK,), jnp.float32),
        pltpu.VMEM((CHUNK, LATENT), jnp.float32),
        pltpu.VMEM((CHUNK, LATENT), jnp.float32),
        pltpu.VMEM((CHUNK, LATENT), jnp.float32),
        pltpu.VMEM_SHARED((NPAD, LATENT), jnp.float32),
        pltpu.VMEM_SHARED((NPAD,), jnp.float32),
    ] + [pltpu.SemaphoreType.DMA] * 17,
)
def _sc_edge_aggregate(h_hbm, asrc_hbm, adst_hbm, sd_hbm, zs_hbm, zd_hbm,
                       s_out, d_out,
                       sd1, sd2, ae_a, ae_b, ae_c, be_a, be_b, be_c,
                       wv_a, wv_b, wv_c, rows_a, rows_b, rows_c,
                       s_sh, d_sh,
                       sem_aa, sem_ab, sem_ac, sem_ba, sem_bb, sem_bc,
                       sem_ra, sem_rb, sem_rc, sem_sa, sem_sb, sem_sc,
                       sem_wa, sem_wb, sem_wc, sem_sd1, sem_sd2):
    c = lax.axis_index("c")
    s = lax.axis_index("s")
    wid = c * NS + s
    row0 = s * ROWS_PT

    bufs_list = [
        (ae_a, be_a, wv_a, rows_a, sem_aa, sem_ba, sem_ra, sem_sa, sem_wa),
        (ae_b, be_b, wv_b, rows_b, sem_ab, sem_bb, sem_rb, sem_sb, sem_wb),
        (ae_c, be_c, wv_c, rows_c, sem_ac, sem_bc, sem_rc, sem_sc, sem_wc),
    ]

    def _issue_g(sd, i, bufs):
        ae, be, wv, rows, sa, sb, sr, ss, sw = bufs
        pltpu.async_copy(asrc_hbm.at[sd.at[i, 0]], ae, sa)
        pltpu.async_copy(adst_hbm.at[sd.at[i, 1]], be, sb)
        pltpu.async_copy(h_hbm.at[sd.at[i, 0]], rows, sr)

    def _compute(sd, i, bufs):
        ae, be, wv, rows, sa, sb, sr, ss, sw = bufs
        pltpu.make_async_copy(asrc_hbm.at[sd.at[i, 0]], ae, sa).wait()
        pltpu.make_async_copy(adst_hbm.at[sd.at[i, 1]], be, sb).wait()
        for g in range(CHUNK // 16):
            sl = pl.ds(g * 16, 16)
            a = ae[sl] + be[sl]
            a = jnp.where(a >= 0.0, a, 0.2 * a)
            wv[sl] = jnp.exp(a)
        pltpu.make_async_copy(h_hbm.at[sd.at[i, 0]], rows, sr).wait()

        def scale16(q, carry2):
            wgrp = wv[pl.ds(q * 16, 16)]
            r0 = q * 16
            for rr in range(16):
                wb = _lane_bcast(wgrp, rr)
                for g2 in range(LATENT // 16):
                    sl2 = pl.ds(g2 * 16, 16)
                    rows[r0 + rr, sl2] = rows[r0 + rr, sl2] * wb
            return carry2

        lax.fori_loop(0, CHUNK // 16, scale16, 0)

    def _issue_s(sd, i, bufs):
        ae, be, wv, rows, sa, sb, sr, ss, sw = bufs
        pltpu.async_copy(rows, s_sh.at[sd.at[i, 1]], ss, add=True)
        pltpu.async_copy(wv, d_sh.at[sd.at[i, 1]], sw, add=True)

    def _wait_s(sd, i, bufs):
        ae, be, wv, rows, sa, sb, sr, ss, sw = bufs
        pltpu.make_async_copy(rows, s_sh.at[sd.at[i, 1]], ss).wait()
        pltpu.make_async_copy(wv, d_sh.at[sd.at[i, 1]], sw).wait()

    pltpu.sync_copy(zs_hbm.at[pl.ds(row0, ROWS_PT)],
                    s_sh.at[pl.ds(row0, ROWS_PT)])

    @pl.when(s == 0)
    def _():
        pltpu.sync_copy(zd_hbm, d_sh)

    plsc.subcore_barrier()

    pltpu.sync_copy(sd_hbm.at[wid, pl.ds(0, 3)], sd1)
    for i in range(3):
        _issue_g(sd1, i, bufs_list[i])

    def body(k, carry):
        base = 6 * k
        pltpu.async_copy(sd_hbm.at[wid, pl.ds(base + 3, 3)], sd2, sem_sd2)
        for i in range(3):
            bufs = bufs_list[i]
            _compute(sd1, i, bufs)
            _issue_s(sd1, i, bufs)
            if i == 0:
                pltpu.make_async_copy(
                    sd_hbm.at[wid, pl.ds(base + 3, 3)], sd2, sem_sd2).wait()
            _wait_s(sd1, i, bufs)
            _issue_g(sd2, i, bufs)
        pltpu.async_copy(sd_hbm.at[wid, pl.ds(base + 6, 3)], sd1, sem_sd1)
        for i in range(3):
            bufs = bufs_list[i]
            _compute(sd2, i, bufs)
            _issue_s(sd2, i, bufs)
            if i == 0:
                pltpu.make_async_copy(
                    sd_hbm.at[wid, pl.ds(base + 6, 3)], sd1, sem_sd1).wait()
            _wait_s(sd2, i, bufs)
            _issue_g(sd1, i, bufs)
        return carry

    lax.fori_loop(0, NBODY, body, 0)

    for i in range(3):
        bufs = bufs_list[i]
        _compute(sd1, i, bufs)
        _issue_s(sd1, i, bufs)
        _wait_s(sd1, i, bufs)
    pltpu.sync_copy(sd_hbm.at[wid, pl.ds(NCHUNKS - 2, 2)], sd2.at[pl.ds(0, 2)])
    _issue_g(sd2, 0, bufs_list[0])
    _issue_g(sd2, 1, bufs_list[1])
    for i in range(2):
        bufs = bufs_list[i]
        _compute(sd2, i, bufs)
        _issue_s(sd2, i, bufs)
        _wait_s(sd2, i, bufs)

    plsc.subcore_barrier()

    pltpu.sync_copy(s_sh.at[pl.ds(row0, ROWS_PT)],
                    s_out.at[c, pl.ds(row0, ROWS_PT)])

    @pl.when(s == 0)
    def _():
        pltpu.sync_copy(d_sh, d_out.at[c])


def _lin_body(x_ref, w_ref, att_ref, h_ref, ab_ref):
    h = jnp.dot(x_ref[...], w_ref[...], preferred_element_type=jnp.float32)
    h_ref[...] = h
    ab_ref[...] = jnp.dot(h, att_ref[...], preferred_element_type=jnp.float32)


def _lin_call(x, W, att):
    return pl.pallas_call(
        _lin_body,
        grid=(GRID_R,),
        in_specs=[
            pl.BlockSpec((BR, LATENT), lambda i: (i, 0)),
            pl.BlockSpec((LATENT, LATENT), lambda i: (0, 0)),
            pl.BlockSpec((LATENT, 2), lambda i: (0, 0)),
        ],
        out_specs=[
            pl.BlockSpec((BR, LATENT), lambda i: (i, 0)),
            pl.BlockSpec((BR, 2), lambda i: (i, 0)),
        ],
        out_shape=[
            jax.ShapeDtypeStruct((N_NODES, LATENT), jnp.float32),
            jax.ShapeDtypeStruct((N_NODES, 2), jnp.float32),
        ],
    )(x, W, att)


def _combine(s_ref, dt_ref, h_ref, ab_ref, bias_ref):
    ab = ab_ref[...]
    e = ab[:, 0:1] + ab[:, 1:2]
    e = jnp.where(e >= 0.0, e, 0.2 * e)
    wself = jnp.exp(e)
    h = h_ref[...]
    ssum = s_ref[0] + s_ref[1] + wself * h
    dt = dt_ref[...]
    den = dt[:, 0:1] + dt[:, 1:2] + wself
    out = ssum / den + bias_ref[...]
    return jnp.where(out > 0.0, out, jnp.exp(out) - 1.0)


def _mid_body(s_ref, dt_ref, h_ref, ab_ref, bias_ref, w2_ref, att2_ref,
              z_ref, h2_ref, ab2_ref):
    z = _combine(s_ref, dt_ref, h_ref, ab_ref, bias_ref)
    z_ref[...] = z
    h2 = jnp.dot(z, w2_ref[...], preferred_element_type=jnp.float32)
    h2_ref[...] = h2
    ab2_ref[...] = jnp.dot(h2, att2_ref[...], preferred_element_type=jnp.float32)


def _mid_call(S, Dt, h, ab, bias, W2, att2):
    return pl.pallas_call(
        _mid_body,
        grid=(GRID_R,),
        in_specs=[
            pl.BlockSpec((NC, BR, LATENT), lambda i: (0, i, 0)),
            pl.BlockSpec((BR, NC), lambda i: (i, 0)),
            pl.BlockSpec((BR, LATENT), lambda i: (i, 0)),
            pl.BlockSpec((BR, 2), lambda i: (i, 0)),
            pl.BlockSpec((1, LATENT), lambda i: (0, 0)),
            pl.BlockSpec((LATENT, LATENT), lambda i: (0, 0)),
            pl.BlockSpec((LATENT, 2), lambda i: (0, 0)),
        ],
        out_specs=[
            pl.BlockSpec((BR, LATENT), lambda i: (i, 0)),
            pl.BlockSpec((BR, LATENT), lambda i: (i, 0)),
            pl.BlockSpec((BR, 2), lambda i: (i, 0)),
        ],
        out_shape=[
            jax.ShapeDtypeStruct((N_NODES, LATENT), jnp.float32),
            jax.ShapeDtypeStruct((N_NODES, LATENT), jnp.float32),
            jax.ShapeDtypeStruct((N_NODES, 2), jnp.float32),
        ],
    )(S, Dt, h, ab, bias, W2, att2)


def _fin_body(s_ref, dt_ref, h_ref, ab_ref, bias_ref, out_ref):
    out_ref[...] = _combine(s_ref, dt_ref, h_ref, ab_ref, bias_ref)


def _fin_call(S, Dt, h, ab, bias):
    return pl.pallas_call(
        _fin_body,
        grid=(GRID_R,),
        in_specs=[
            pl.BlockSpec((NC, BR, LATENT), lambda i: (0, i, 0)),
            pl.BlockSpec((BR, NC), lambda i: (i, 0)),
            pl.BlockSpec((BR, LATENT), lambda i: (i, 0)),
            pl.BlockSpec((BR, 2), lambda i: (i, 0)),
            pl.BlockSpec((1, LATENT), lambda i: (0, 0)),
        ],
        out_specs=pl.BlockSpec((BR, LATENT), lambda i: (i, 0)),
        out_shape=jax.ShapeDtypeStruct((N_NODES, LATENT), jnp.float32),
    )(S, Dt, h, ab, bias)


def kernel(x, edge_index, W1, att_src1, att_dst1, bias1,
           W2, att_src2, att_dst2, bias2):
    ei = edge_index.astype(jnp.int32)
    sd_rs = jnp.stack([ei[0].reshape(NW, NCHUNKS, CHUNK),
                       ei[1].reshape(NW, NCHUNKS, CHUNK)], axis=2)
    att1 = jnp.stack([att_src1, att_dst1], axis=1)
    att2 = jnp.stack([att_src2, att_dst2], axis=1)
    b1 = bias1.reshape(1, LATENT)
    b2 = bias2.reshape(1, LATENT)
    zs = jnp.zeros((NPAD, LATENT), jnp.float32)
    zd = jnp.zeros((NPAD,), jnp.float32)

    h1, ab1 = _lin_call(x, W1, att1)
    asrc1 = ab1[:, 0].ravel()
    adst1 = ab1[:, 1].ravel()
    S1, Dn1 = _sc_edge_aggregate(h1, asrc1, adst1, sd_rs, zs, zd)
    z, h2, ab2 = _mid_call(S1, Dn1.T, h1, ab1, b1, W2, att2)
    asrc2 = ab2[:, 0].ravel()
    adst2 = ab2[:, 1].ravel()
    S2, Dn2 = _sc_edge_aggregate(h2, asrc2, adst2, sd_rs, zs, zd)
    xbar = _fin_call(S2, Dn2.T, h2, ab2, b2)
    return (xbar, z)


# speedup vs baseline: 57.9741x; 1.0051x over previous
import functools

import jax
import jax.numpy as jnp
from jax import lax
from jax.experimental import pallas as pl
from jax.experimental.pallas import tpu as pltpu
from jax.experimental.pallas import tpu_sc as plsc

N_NODES = 10000
LATENT = 128
N_EDGES = 320000
NC = 2
NS = 16
NW = NC * NS
EPT = N_EDGES // NW
CHUNK = 80
NCHUNKS = EPT // CHUNK
NBODY = 20
NPAD = 10240
ROWS_PT = NPAD // NS
BR = 2000
GRID_R = N_NODES // BR

_mesh = plsc.VectorSubcoreMesh(core_axis_name="c", subcore_axis_name="s")

_BCAST_DNUMS = lax.GatherDimensionNumbers(
    offset_dims=(), collapsed_slice_dims=(0,), start_index_map=(0,))


def _lane_bcast(vec16, lane):
    idx = jnp.full((16, 1), lane, dtype=jnp.int32)
    return lax.gather(vec16, idx, _BCAST_DNUMS, (1,),
                      mode=lax.GatherScatterMode.PROMISE_IN_BOUNDS)


@functools.partial(
    pl.kernel,
    mesh=_mesh,
    compiler_params=pltpu.CompilerParams(needs_layout_passes=False),
    out_type=[
        jax.ShapeDtypeStruct((NC, NPAD, LATENT), jnp.float32),
        jax.ShapeDtypeStruct((NC, NPAD), jnp.float32),
    ],
    scratch_types=[
        pltpu.VMEM((3, 1, CHUNK), jnp.int32),
        pltpu.VMEM((3, 1, CHUNK), jnp.int32),
        pltpu.VMEM((3, 1, CHUNK), jnp.int32),
        pltpu.VMEM((3, 1, CHUNK), jnp.int32),
        pltpu.VMEM((CHUNK,), jnp.float32),
        pltpu.VMEM((CHUNK,), jnp.float32),
        pltpu.VMEM((CHUNK,), jnp.float32),
        pltpu.VMEM((CHUNK,), jnp.float32),
        pltpu.VMEM((CHUNK,), jnp.float32),
        pltpu.VMEM((CHUNK,), jnp.float32),
        pltpu.VMEM((CHUNK,), jnp.float32),
        pltpu.VMEM((CHUNK,), jnp.float32),
        pltpu.VMEM((CHUNK,), jnp.float32),
        pltpu.VMEM((CHUNK, LATENT), jnp.float32),
        pltpu.VMEM((CHUNK, LATENT), jnp.float32),
        pltpu.VMEM((CHUNK, LATENT), jnp.float32),
        pltpu.VMEM_SHARED((NPAD, LATENT), jnp.float32),
        pltpu.VMEM_SHARED((NPAD,), jnp.float32),
    ] + [pltpu.SemaphoreType.DMA] * 19,
)
def _sc_edge_aggregate(h_hbm, asrc_hbm, adst_hbm, src_hbm, dst_hbm,
                       s_out, d_out,
                       sg1, sg2, dg1, dg2,
                       ae_a, ae_b, ae_c, be_a, be_b, be_c,
                       wv_a, wv_b, wv_c, rows_a, rows_b, rows_c,
                       s_sh, d_sh,
                       sem_aa, sem_ab, sem_ac, sem_ba, sem_bb, sem_bc,
                       sem_ra, sem_rb, sem_rc, sem_sa, sem_sb, sem_sc,
                       sem_wa, sem_wb, sem_wc,
                       sem_g1s, sem_g1d, sem_g2s, sem_g2d):
    c = lax.axis_index("c")
    s = lax.axis_index("s")
    wid = c * NS + s
    row0 = s * ROWS_PT

    bufs_list = [
        (ae_a, be_a, wv_a, rows_a, sem_aa, sem_ba, sem_ra, sem_sa, sem_wa),
        (ae_b, be_b, wv_b, rows_b, sem_ab, sem_bb, sem_rb, sem_sb, sem_wb),
        (ae_c, be_c, wv_c, rows_c, sem_ac, sem_bc, sem_rc, sem_sc, sem_wc),
    ]

    def _issue_g(sg, dg, i, bufs):
        ae, be, wv, rows, sa, sb, sr, ss, sw = bufs
        pltpu.async_copy(asrc_hbm.at[sg.at[i, 0]], ae, sa)
        pltpu.async_copy(adst_hbm.at[dg.at[i, 0]], be, sb)
        pltpu.async_copy(h_hbm.at[sg.at[i, 0]], rows, sr)

    def _compute(sg, dg, i, bufs):
        ae, be, wv, rows, sa, sb, sr, ss, sw = bufs
        pltpu.make_async_copy(asrc_hbm.at[sg.at[i, 0]], ae, sa).wait()
        pltpu.make_async_copy(adst_hbm.at[dg.at[i, 0]], be, sb).wait()
        for g in range(CHUNK // 16):
            sl = pl.ds(g * 16, 16)
            a = ae[sl] + be[sl]
            a = jnp.where(a >= 0.0, a, 0.2 * a)
            wv[sl] = jnp.exp(a)
        pltpu.make_async_copy(h_hbm.at[sg.at[i, 0]], rows, sr).wait()

        def scale16(q, carry2):
            wgrp = wv[pl.ds(q * 16, 16)]
            r0 = q * 16
            for rr in range(16):
                wb = _lane_bcast(wgrp, rr)
                for g2 in range(LATENT // 16):
                    sl2 = pl.ds(g2 * 16, 16)
                    rows[r0 + rr, sl2] = rows[r0 + rr, sl2] * wb
            return carry2

        lax.fori_loop(0, CHUNK // 16, scale16, 0)

    def _issue_s(dg, i, bufs):
        ae, be, wv, rows, sa, sb, sr, ss, sw = bufs
        pltpu.async_copy(rows, s_sh.at[dg.at[i, 0]], ss, add=True)
        pltpu.async_copy(wv, d_sh.at[dg.at[i, 0]], sw, add=True)

    def _wait_s(dg, i, bufs):
        ae, be, wv, rows, sa, sb, sr, ss, sw = bufs
        pltpu.make_async_copy(rows, s_sh.at[dg.at[i, 0]], ss).wait()
        pltpu.make_async_copy(wv, d_sh.at[dg.at[i, 0]], sw).wait()

    def _zrow(r, carry):
        for g2 in range(LATENT // 16):
            rows_a[r, pl.ds(g2 * 16, 16)] = jnp.zeros((16,), jnp.float32)
        return carry

    lax.fori_loop(0, CHUNK, _zrow, 0)
    for g in range(CHUNK // 16):
        wv_a[pl.ds(g * 16, 16)] = jnp.zeros((16,), jnp.float32)
    for t in range(ROWS_PT // CHUNK):
        pltpu.async_copy(rows_a, s_sh.at[pl.ds(row0 + t * CHUNK, CHUNK)],
                         sem_sa)
        pltpu.async_copy(wv_a, d_sh.at[pl.ds(row0 + t * CHUNK, CHUNK)],
                         sem_wa)
    for t in range(ROWS_PT // CHUNK):
        pltpu.make_async_copy(rows_a,
                              s_sh.at[pl.ds(row0 + t * CHUNK, CHUNK)],
                              sem_sa).wait()
        pltpu.make_async_copy(wv_a,
                              d_sh.at[pl.ds(row0 + t * CHUNK, CHUNK)],
                              sem_wa).wait()

    plsc.subcore_barrier()

    pltpu.sync_copy(src_hbm.at[wid, pl.ds(0, 3)], sg1)
    pltpu.sync_copy(dst_hbm.at[wid, pl.ds(0, 3)], dg1)
    _issue_g(sg1, dg1, 0, bufs_list[0])
    _issue_g(sg1, dg1, 1, bufs_list[1])

    def body(k, carry):
        base = 6 * k
        _compute(sg1, dg1, 0, bufs_list[0])
        _issue_s(dg1, 0, bufs_list[0])

        @pl.when(k > 0)
        def _():
            _wait_s(dg2, 2, bufs_list[2])
        pltpu.async_copy(src_hbm.at[wid, pl.ds(base + 3, 3)], sg2, sem_g2s)
        pltpu.async_copy(dst_hbm.at[wid, pl.ds(base + 3, 3)], dg2, sem_g2d)
        _issue_g(sg1, dg1, 2, bufs_list[2])
        _compute(sg1, dg1, 1, bufs_list[1])
        _issue_s(dg1, 1, bufs_list[1])
        pltpu.make_async_copy(
            src_hbm.at[wid, pl.ds(base + 3, 3)], sg2, sem_g2s).wait()
        pltpu.make_async_copy(
            dst_hbm.at[wid, pl.ds(base + 3, 3)], dg2, sem_g2d).wait()
        _wait_s(dg1, 0, bufs_list[0])
        _issue_g(sg2, dg2, 0, bufs_list[0])
        _compute(sg1, dg1, 2, bufs_list[2])
        _issue_s(dg1, 2, bufs_list[2])
        _wait_s(dg1, 1, bufs_list[1])
        _issue_g(sg2, dg2, 1, bufs_list[1])
        _compute(sg2, dg2, 0, bufs_list[0])
        _issue_s(dg2, 0, bufs_list[0])
        _wait_s(dg1, 2, bufs_list[2])
        _issue_g(sg2, dg2, 2, bufs_list[2])
        pltpu.async_copy(src_hbm.at[wid, pl.ds(base + 6, 3)], sg1, sem_g1s)
        pltpu.async_copy(dst_hbm.at[wid, pl.ds(base + 6, 3)], dg1, sem_g1d)
        _compute(sg2, dg2, 1, bufs_list[1])
        _issue_s(dg2, 1, bufs_list[1])
        pltpu.make_async_copy(
            src_hbm.at[wid, pl.ds(base + 6, 3)], sg1, sem_g1s).wait()
        pltpu.make_async_copy(
            dst_hbm.at[wid, pl.ds(base + 6, 3)], dg1, sem_g1d).wait()
        _wait_s(dg2, 0, bufs_list[0])
        _issue_g(sg1, dg1, 0, bufs_list[0])
        _compute(sg2, dg2, 2, bufs_list[2])
        _issue_s(dg2, 2, bufs_list[2])
        _wait_s(dg2, 1, bufs_list[1])
        _issue_g(sg1, dg1, 1, bufs_list[1])
        return carry

    lax.fori_loop(0, NBODY, body, 0)

    _wait_s(dg2, 2, bufs_list[2])
    _issue_g(sg1, dg1, 2, bufs_list[2])
    _compute(sg1, dg1, 0, bufs_list[0])
    _issue_s(dg1, 0, bufs_list[0])
    _compute(sg1, dg1, 1, bufs_list[1])
    _issue_s(dg1, 1, bufs_list[1])
    _wait_s(dg1, 0, bufs_list[0])
    _compute(sg1, dg1, 2, bufs_list[2])
    _issue_s(dg1, 2, bufs_list[2])
    _wait_s(dg1, 1, bufs_list[1])
    _wait_s(dg1, 2, bufs_list[2])
    pltpu.sync_copy(src_hbm.at[wid, pl.ds(NCHUNKS - 2, 2)],
                    sg2.at[pl.ds(0, 2)])
    pltpu.sync_copy(dst_hbm.at[wid, pl.ds(NCHUNKS - 2, 2)],
                    dg2.at[pl.ds(0, 2)])
    _issue_g(sg2, dg2, 0, bufs_list[0])
    _issue_g(sg2, dg2, 1, bufs_list[1])
    _compute(sg2, dg2, 0, bufs_list[0])
    _issue_s(dg2, 0, bufs_list[0])
    _compute(sg2, dg2, 1, bufs_list[1])
    _issue_s(dg2, 1, bufs_list[1])
    _wait_s(dg2, 0, bufs_list[0])
    _wait_s(dg2, 1, bufs_list[1])

    plsc.subcore_barrier()

    pltpu.sync_copy(s_sh.at[pl.ds(row0, ROWS_PT)],
                    s_out.at[c, pl.ds(row0, ROWS_PT)])
    pltpu.sync_copy(d_sh.at[pl.ds(row0, ROWS_PT)],
                    d_out.at[c, pl.ds(row0, ROWS_PT)])


def _lin_body(x_ref, w_ref, as_ref, ad_ref, h_ref, a_ref, b_ref):
    h = jnp.dot(x_ref[...], w_ref[...], preferred_element_type=jnp.float32)
    h_ref[...] = h
    a_ref[...] = jnp.dot(h, as_ref[...], preferred_element_type=jnp.float32)
    b_ref[...] = jnp.dot(h, ad_ref[...], preferred_element_type=jnp.float32)


def _lin_call(x, W, att_s, att_d):
    return pl.pallas_call(
        _lin_body,
        grid=(GRID_R,),
        in_specs=[
            pl.BlockSpec((BR, LATENT), lambda i: (i, 0)),
            pl.BlockSpec((LATENT, LATENT), lambda i: (0, 0)),
            pl.BlockSpec((LATENT, 1), lambda i: (0, 0)),
            pl.BlockSpec((LATENT, 1), lambda i: (0, 0)),
        ],
        out_specs=[
            pl.BlockSpec((BR, LATENT), lambda i: (i, 0)),
            pl.BlockSpec((BR, 1), lambda i: (i, 0)),
            pl.BlockSpec((BR, 1), lambda i: (i, 0)),
        ],
        out_shape=[
            jax.ShapeDtypeStruct((N_NODES, LATENT), jnp.float32),
            jax.ShapeDtypeStruct((N_NODES, 1), jnp.float32),
            jax.ShapeDtypeStruct((N_NODES, 1), jnp.float32),
        ],
    )(x, W, att_s, att_d)


def _combine(s_ref, dt_ref, h_ref, a_ref, b_ref, bias_ref):
    e = a_ref[...] + b_ref[...]
    e = jnp.where(e >= 0.0, e, 0.2 * e)
    wself = jnp.exp(e)
    h = h_ref[...]
    ssum = s_ref[0] + s_ref[1] + wself * h
    dt = dt_ref[...]
    den = dt[0] + dt[1] + wself
    out = ssum / den + bias_ref[...]
    return jnp.where(out > 0.0, out, jnp.exp(out) - 1.0)


def _mid_body(s_ref, dt_ref, h_ref, a_ref, b_ref, bias_ref, w2_ref,
              as2_ref, ad2_ref, z_ref, h2_ref, a2_ref, b2_ref):
    z = _combine(s_ref, dt_ref, h_ref, a_ref, b_ref, bias_ref)
    z_ref[...] = z
    h2 = jnp.dot(z, w2_ref[...], preferred_element_type=jnp.float32)
    h2_ref[...] = h2
    a2_ref[...] = jnp.dot(h2, as2_ref[...], preferred_element_type=jnp.float32)
    b2_ref[...] = jnp.dot(h2, ad2_ref[...], preferred_element_type=jnp.float32)


def _mid_call(S, Dt, h, a, b, bias, W2, att_s2, att_d2):
    return pl.pallas_call(
        _mid_body,
        grid=(GRID_R,),
        in_specs=[
            pl.BlockSpec((NC, BR, LATENT), lambda i: (0, i, 0)),
            pl.BlockSpec((NC, BR, 1), lambda i: (0, i, 0)),
            pl.BlockSpec((BR, LATENT), lambda i: (i, 0)),
            pl.BlockSpec((BR, 1), lambda i: (i, 0)),
            pl.BlockSpec((BR, 1), lambda i: (i, 0)),
            pl.BlockSpec((1, LATENT), lambda i: (0, 0)),
            pl.BlockSpec((LATENT, LATENT), lambda i: (0, 0)),
            pl.BlockSpec((LATENT, 1), lambda i: (0, 0)),
            pl.BlockSpec((LATENT, 1), lambda i: (0, 0)),
        ],
        out_specs=[
            pl.BlockSpec((BR, LATENT), lambda i: (i, 0)),
            pl.BlockSpec((BR, LATENT), lambda i: (i, 0)),
            pl.BlockSpec((BR, 1), lambda i: (i, 0)),
            pl.BlockSpec((BR, 1), lambda i: (i, 0)),
        ],
        out_shape=[
            jax.ShapeDtypeStruct((N_NODES, LATENT), jnp.float32),
            jax.ShapeDtypeStruct((N_NODES, LATENT), jnp.float32),
            jax.ShapeDtypeStruct((N_NODES, 1), jnp.float32),
            jax.ShapeDtypeStruct((N_NODES, 1), jnp.float32),
        ],
    )(S, Dt, h, a, b, bias, W2, att_s2, att_d2)


def _fin_body(s_ref, dt_ref, h_ref, a_ref, b_ref, bias_ref, out_ref):
    out_ref[...] = _combine(s_ref, dt_ref, h_ref, a_ref, b_ref, bias_ref)


def _fin_call(S, Dt, h, a, b, bias):
    return pl.pallas_call(
        _fin_body,
        grid=(GRID_R,),
        in_specs=[
            pl.BlockSpec((NC, BR, LATENT), lambda i: (0, i, 0)),
            pl.BlockSpec((NC, BR, 1), lambda i: (0, i, 0)),
            pl.BlockSpec((BR, LATENT), lambda i: (i, 0)),
            pl.BlockSpec((BR, 1), lambda i: (i, 0)),
            pl.BlockSpec((BR, 1), lambda i: (i, 0)),
            pl.BlockSpec((1, LATENT), lambda i: (0, 0)),
        ],
        out_specs=pl.BlockSpec((BR, LATENT), lambda i: (i, 0)),
        out_shape=jax.ShapeDtypeStruct((N_NODES, LATENT), jnp.float32),
    )(S, Dt, h, a, b, bias)


def kernel(x, edge_index, W1, att_src1, att_dst1, bias1,
           W2, att_src2, att_dst2, bias2):
    ei = edge_index.astype(jnp.int32)
    src_rs = ei[0].reshape(NW, NCHUNKS, 1, CHUNK)
    dst_rs = ei[1].reshape(NW, NCHUNKS, 1, CHUNK)
    as1 = att_src1.reshape(LATENT, 1)
    ad1 = att_dst1.reshape(LATENT, 1)
    as2 = att_src2.reshape(LATENT, 1)
    ad2 = att_dst2.reshape(LATENT, 1)
    b1 = bias1.reshape(1, LATENT)
    b2 = bias2.reshape(1, LATENT)

    h1, a1, bb1 = _lin_call(x, W1, as1, ad1)
    S1, Dn1 = _sc_edge_aggregate(h1, a1.reshape(N_NODES), bb1.reshape(N_NODES),
                                 src_rs, dst_rs)
    z, h2, a2, bb2 = _mid_call(S1, Dn1.reshape(NC, NPAD, 1), h1, a1, bb1, b1,
                               W2, as2, ad2)
    S2, Dn2 = _sc_edge_aggregate(h2, a2.reshape(N_NODES), bb2.reshape(N_NODES),
                                 src_rs, dst_rs)
    xbar = _fin_call(S2, Dn2.reshape(NC, NPAD, 1), h2, a2, bb2, b2)
    return (xbar, z)
